# baseline (speedup 1.0000x reference)
import functools
import math

import jax
import jax.numpy as jnp
from jax import lax
from jax.experimental import pallas as pl
from jax.experimental.pallas import tpu as pltpu
from jax.experimental.pallas import tpu_sc as plsc

NUM_HEADS = 12
NUM_BUCKETS = 32
MAX_DISTANCE = 128
QLEN = 2048
KLEN = 2048
SHIFTS = 16
DIAG_LANES = 4080
NUM_WORKERS = 32
GROUPS = (NUM_HEADS * QLEN) // 8
GROUPS_PER_WORKER = GROUPS // NUM_WORKERS
GROUPS_PER_HEAD = QLEN // 8
_HEAD_WORDS = SHIFTS * DIAG_LANES


def _diag_table_kernel(rel_bias_ref, out_ref):
    s = lax.broadcasted_iota(jnp.int32, (SHIFTS, DIAG_LANES), 0)
    u = lax.broadcasted_iota(jnp.int32, (SHIFTS, DIAG_LANES), 1)
    t = u + s
    n = (QLEN - 1) - t
    half = NUM_BUCKETS // 2
    max_exact = half // 2
    ret = jnp.where(n < 0, half, 0)
    na = jnp.abs(n)
    is_small = na < max_exact
    nf = jnp.maximum(na.astype(jnp.float32), 1.0) / max_exact
    val_if_large = max_exact + (
        jnp.log(nf) / math.log(MAX_DISTANCE / max_exact) * (half - max_exact)
    ).astype(jnp.int32)
    val_if_large = jnp.minimum(val_if_large, half - 1)
    bucket = ret + jnp.where(is_small, na, val_if_large)
    for h in range(NUM_HEADS):
        acc = jnp.zeros((SHIFTS, DIAG_LANES), jnp.float32)
        for b in range(NUM_BUCKETS):
            acc = jnp.where(bucket == b, rel_bias_ref[b, h], acc)
        out_ref[h] = acc


def _assemble_group(diag_v, stage, g):
    gh = g & (GROUPS_PER_HEAD - 1)
    st0 = (QLEN - 1) - gh * 8
    for sl in range(8):
        start = st0 - sl
        sft = start & (SHIFTS - 1)
        base = start - sft
        row_off = sft * DIAG_LANES + base

        def chunk_body(cc, carry):
            off = pl.multiple_of(row_off + cc * 128, SHIFTS)
            col = cc * 128
            for u in range(8):
                stage[sl, pl.ds(col + u * 16, 16)] = diag_v[
                    pl.ds(pl.multiple_of(off + u * 16, SHIFTS), 16)
                ]
            return carry

        lax.fori_loop(0, KLEN // 128, chunk_body, 0)


def _fanout_body(diag_hbm, out_hbm, diag_v, stage0, stage1, sem):
    wid = lax.axis_index("s") * 2 + lax.axis_index("c")
    g_lo = wid * GROUPS_PER_WORKER
    h_lo = g_lo // GROUPS_PER_HEAD
    h_hi = (g_lo + GROUPS_PER_WORKER - 1) // GROUPS_PER_HEAD

    def head_body(h, carry):
        g0 = jnp.maximum(g_lo, h * GROUPS_PER_HEAD)
        g1 = jnp.minimum(g_lo + GROUPS_PER_WORKER, (h + 1) * GROUPS_PER_HEAD)
        pltpu.sync_copy(diag_hbm.at[pl.ds(h * _HEAD_WORDS, _HEAD_WORDS)], diag_v)

        def pair_body(p, carry2):
            ga = 2 * p
            gb = 2 * p + 1
            _assemble_group(diag_v, stage0, ga)
            cp_a = pltpu.make_async_copy(
                stage0, out_hbm.at[pl.ds(8 * ga, 8), :], sem
            )
            cp_a.start()
            _assemble_group(diag_v, stage1, gb)
            cp_b = pltpu.make_async_copy(
                stage1, out_hbm.at[pl.ds(8 * gb, 8), :], sem
            )
            cp_b.start()
            cp_a.wait()
            cp_b.wait()
            return carry2

        lax.fori_loop(g0 >> 1, g1 >> 1, pair_body, 0)
        return carry

    lax.fori_loop(h_lo, h_hi + 1, head_body, 0)


def kernel(query, key, rel_bias):
    batch_size = query.shape[0]

    diag16 = pl.pallas_call(
        _diag_table_kernel,
        out_shape=jax.ShapeDtypeStruct((NUM_HEADS, SHIFTS, DIAG_LANES), jnp.float32),
    )(rel_bias)

    fanout = pl.kernel(
        _fanout_body,
        out_type=jax.ShapeDtypeStruct((NUM_HEADS * QLEN, KLEN), jnp.float32),
        mesh=plsc.VectorSubcoreMesh(core_axis_name="c", subcore_axis_name="s"),
        scratch_types=[
            pltpu.VMEM((_HEAD_WORDS,), jnp.float32),
            pltpu.VMEM((8, KLEN), jnp.float32),
            pltpu.VMEM((8, KLEN), jnp.float32),
            pltpu.SemaphoreType.DMA,
        ],
    )
    out_flat = fanout(diag16.reshape(NUM_HEADS * _HEAD_WORDS))
    out = out_flat.reshape(1, NUM_HEADS, QLEN, KLEN)
    return jnp.broadcast_to(out, (batch_size, NUM_HEADS, QLEN, KLEN))


# speedup vs baseline: 1.1078x; 1.1078x over previous
import functools
import math

import jax
import jax.numpy as jnp
from jax import lax
from jax.experimental import pallas as pl
from jax.experimental.pallas import tpu as pltpu
from jax.experimental.pallas import tpu_sc as plsc

NUM_HEADS = 12
NUM_BUCKETS = 32
MAX_DISTANCE = 128
QLEN = 2048
KLEN = 2048
SHIFTS = 16
DIAG_LANES = 4096
NUM_WORKERS = 32
GROUPS = (NUM_HEADS * QLEN) // 8
GROUPS_PER_WORKER = GROUPS // NUM_WORKERS
GROUPS_PER_HEAD = QLEN // 8


def _diag_table_kernel(rel_bias_ref, out_ref):
    s = lax.broadcasted_iota(jnp.int32, (SHIFTS, DIAG_LANES), 0)
    u = lax.broadcasted_iota(jnp.int32, (SHIFTS, DIAG_LANES), 1)
    t = u + s
    n = (QLEN - 1) - t
    half = NUM_BUCKETS // 2
    max_exact = half // 2
    ret = jnp.where(n < 0, half, 0)
    na = jnp.abs(n)
    is_small = na < max_exact
    nf = jnp.maximum(na.astype(jnp.float32), 1.0) / max_exact
    val_if_large = max_exact + (
        jnp.log(nf) / math.log(MAX_DISTANCE / max_exact) * (half - max_exact)
    ).astype(jnp.int32)
    val_if_large = jnp.minimum(val_if_large, half - 1)
    bucket = ret + jnp.where(is_small, na, val_if_large)
    for h in range(NUM_HEADS):
        acc = jnp.zeros((SHIFTS, DIAG_LANES), jnp.float32)
        for b in range(NUM_BUCKETS):
            acc = jnp.where(bucket == b, rel_bias_ref[b, h], acc)
        out_ref[h] = acc


def _assemble_group(diag_v, stage, g):
    gh = g & (GROUPS_PER_HEAD - 1)
    st0 = (QLEN - 1) - gh * 8
    for sl in range(8):
        start = st0 - sl
        sft = start & (SHIFTS - 1)
        base = start - sft

        def chunk_body(cc, carry):
            col = cc * 128
            src = pl.multiple_of(base + col, SHIFTS)
            for u in range(8):
                stage[sl, pl.ds(col + u * 16, 16)] = diag_v[
                    sft, pl.ds(pl.multiple_of(src + u * 16, SHIFTS), 16)
                ]
            return carry

        lax.fori_loop(0, KLEN // 128, chunk_body, 0)


def _fanout_body(diag_hbm, out_hbm, diag_v, stage0, stage1, sem):
    wid = lax.axis_index("s") * 2 + lax.axis_index("c")
    g_lo = wid * GROUPS_PER_WORKER
    h_lo = g_lo // GROUPS_PER_HEAD
    h_hi = (g_lo + GROUPS_PER_WORKER - 1) // GROUPS_PER_HEAD

    def head_body(h, carry):
        g0 = jnp.maximum(g_lo, h * GROUPS_PER_HEAD)
        g1 = jnp.minimum(g_lo + GROUPS_PER_WORKER, (h + 1) * GROUPS_PER_HEAD)
        pltpu.sync_copy(diag_hbm.at[h], diag_v)

        def pair_body(p, carry2):
            ga = 2 * p
            gb = 2 * p + 1
            _assemble_group(diag_v, stage0, ga)
            cp_a = pltpu.make_async_copy(
                stage0, out_hbm.at[pl.ds(8 * ga, 8), :], sem
            )
            cp_a.start()
            _assemble_group(diag_v, stage1, gb)
            cp_b = pltpu.make_async_copy(
                stage1, out_hbm.at[pl.ds(8 * gb, 8), :], sem
            )
            cp_b.start()
            cp_a.wait()
            cp_b.wait()
            return carry2

        lax.fori_loop(g0 >> 1, g1 >> 1, pair_body, 0)
        return carry

    lax.fori_loop(h_lo, h_hi + 1, head_body, 0)


def kernel(query, key, rel_bias):
    batch_size = query.shape[0]

    diag16 = pl.pallas_call(
        _diag_table_kernel,
        out_shape=jax.ShapeDtypeStruct((NUM_HEADS, SHIFTS, DIAG_LANES), jnp.float32),
    )(rel_bias)

    fanout = pl.kernel(
        _fanout_body,
        out_type=jax.ShapeDtypeStruct((NUM_HEADS * QLEN, KLEN), jnp.float32),
        mesh=plsc.VectorSubcoreMesh(core_axis_name="c", subcore_axis_name="s"),
        scratch_types=[
            pltpu.VMEM((SHIFTS, DIAG_LANES), jnp.float32),
            pltpu.VMEM((8, KLEN), jnp.float32),
            pltpu.VMEM((8, KLEN), jnp.float32),
            pltpu.SemaphoreType.DMA,
        ],
    )
    out_flat = fanout(diag16)
    out = out_flat.reshape(1, NUM_HEADS, QLEN, KLEN)
    return jnp.broadcast_to(out, (batch_size, NUM_HEADS, QLEN, KLEN))


# speedup vs baseline: 2.4776x; 2.2365x over previous
import functools
import math

import jax
import jax.numpy as jnp
from jax import lax
from jax.experimental import pallas as pl
from jax.experimental.pallas import tpu as pltpu
from jax.experimental.pallas import tpu_sc as plsc

NUM_HEADS = 12
NUM_BUCKETS = 32
MAX_DISTANCE = 128
QLEN = 2048
KLEN = 2048
SHIFTS = 16
DIAG_LANES = 4096
NUM_WORKERS = 32
GROUPS = (NUM_HEADS * QLEN) // 8
GROUPS_PER_WORKER = GROUPS // NUM_WORKERS
GROUPS_PER_HEAD = QLEN // 8


def _diag_table_kernel(rel_bias_ref, out_ref):
    s = lax.broadcasted_iota(jnp.int32, (SHIFTS, DIAG_LANES), 0)
    u = lax.broadcasted_iota(jnp.int32, (SHIFTS, DIAG_LANES), 1)
    t = u + s
    n = (QLEN - 1) - t
    half = NUM_BUCKETS // 2
    max_exact = half // 2
    ret = jnp.where(n < 0, half, 0)
    na = jnp.abs(n)
    is_small = na < max_exact
    nf = jnp.maximum(na.astype(jnp.float32), 1.0) / max_exact
    val_if_large = max_exact + (
        jnp.log(nf) / math.log(MAX_DISTANCE / max_exact) * (half - max_exact)
    ).astype(jnp.int32)
    val_if_large = jnp.minimum(val_if_large, half - 1)
    bucket = ret + jnp.where(is_small, na, val_if_large)
    for h in range(NUM_HEADS):
        acc = jnp.zeros((SHIFTS, DIAG_LANES), jnp.float32)
        for b in range(NUM_BUCKETS):
            acc = jnp.where(bucket == b, rel_bias_ref[b, h], acc)
        out_ref[h] = acc


def _assemble_group(diag_v, stage, g):
    gh = g & (GROUPS_PER_HEAD - 1)
    st0 = (QLEN - 1) - gh * 8
    for sl in range(8):
        start = st0 - sl
        sft = start & (SHIFTS - 1)
        base = start - sft

        def chunk_body(cc, carry):
            col = cc * 256
            src = pl.multiple_of(base + col, SHIFTS)
            vals = [
                diag_v[sft, pl.ds(pl.multiple_of(src + u * 16, SHIFTS), 16)]
                for u in range(16)
            ]
            for u in range(16):
                stage[sl, pl.ds(col + u * 16, 16)] = vals[u]
            return carry

        lax.fori_loop(0, KLEN // 256, chunk_body, 0)


def _fanout_body(diag_hbm, out_hbm, diag_v, stage0, stage1, sem):
    wid = lax.axis_index("s") * 2 + lax.axis_index("c")
    g_lo = wid * GROUPS_PER_WORKER
    h_lo = g_lo // GROUPS_PER_HEAD
    h_hi = (g_lo + GROUPS_PER_WORKER - 1) // GROUPS_PER_HEAD

    def head_body(h, carry):
        g0 = jnp.maximum(g_lo, h * GROUPS_PER_HEAD)
        g1 = jnp.minimum(g_lo + GROUPS_PER_WORKER, (h + 1) * GROUPS_PER_HEAD)
        pltpu.sync_copy(diag_hbm.at[h], diag_v)

        def pair_body(p, carry2):
            ga = 2 * p
            gb = 2 * p + 1
            _assemble_group(diag_v, stage0, ga)
            cp_a = pltpu.make_async_copy(
                stage0, out_hbm.at[pl.ds(8 * ga, 8), :], sem
            )
            cp_a.start()
            _assemble_group(diag_v, stage1, gb)
            cp_b = pltpu.make_async_copy(
                stage1, out_hbm.at[pl.ds(8 * gb, 8), :], sem
            )
            cp_b.start()
            cp_a.wait()
            cp_b.wait()
            return carry2

        lax.fori_loop(g0 >> 1, g1 >> 1, pair_body, 0)
        return carry

    lax.fori_loop(h_lo, h_hi + 1, head_body, 0)


def kernel(query, key, rel_bias):
    batch_size = query.shape[0]

    diag16 = pl.pallas_call(
        _diag_table_kernel,
        out_shape=jax.ShapeDtypeStruct((NUM_HEADS, SHIFTS, DIAG_LANES), jnp.float32),
    )(rel_bias)

    fanout = pl.kernel(
        _fanout_body,
        out_type=jax.ShapeDtypeStruct((NUM_HEADS * QLEN, KLEN), jnp.float32),
        mesh=plsc.VectorSubcoreMesh(core_axis_name="c", subcore_axis_name="s"),
        scratch_types=[
            pltpu.VMEM((SHIFTS, DIAG_LANES), jnp.float32),
            pltpu.VMEM((8, KLEN), jnp.float32),
            pltpu.VMEM((8, KLEN), jnp.float32),
            pltpu.SemaphoreType.DMA,
        ],
    )
    out_flat = fanout(diag16)
    out = out_flat.reshape(1, NUM_HEADS, QLEN, KLEN)
    return jnp.broadcast_to(out, (batch_size, NUM_HEADS, QLEN, KLEN))


# speedup vs baseline: 4.2643x; 1.7211x over previous
import functools
import math

import jax
import jax.numpy as jnp
from jax import lax
from jax.experimental import pallas as pl
from jax.experimental.pallas import tpu as pltpu
from jax.experimental.pallas import tpu_sc as plsc

NUM_HEADS = 12
NUM_BUCKETS = 32
MAX_DISTANCE = 128
QLEN = 2048
KLEN = 2048
SHIFTS = 16
DIAG_LANES = 4096
NUM_WORKERS = 32
GROUPS = (NUM_HEADS * QLEN) // 8
GROUPS_PER_WORKER = GROUPS // NUM_WORKERS
GROUPS_PER_HEAD = QLEN // 8


def _diag_table_kernel(rel_bias_ref, out_ref):
    s = lax.broadcasted_iota(jnp.int32, (SHIFTS, DIAG_LANES), 0)
    u = lax.broadcasted_iota(jnp.int32, (SHIFTS, DIAG_LANES), 1)
    t = u + s
    n = (QLEN - 1) - t
    half = NUM_BUCKETS // 2
    max_exact = half // 2
    ret = jnp.where(n < 0, half, 0)
    na = jnp.abs(n)
    is_small = na < max_exact
    nf = jnp.maximum(na.astype(jnp.float32), 1.0) / max_exact
    val_if_large = max_exact + (
        jnp.log(nf) / math.log(MAX_DISTANCE / max_exact) * (half - max_exact)
    ).astype(jnp.int32)
    val_if_large = jnp.minimum(val_if_large, half - 1)
    bucket = ret + jnp.where(is_small, na, val_if_large)
    for h in range(NUM_HEADS):
        acc = jnp.zeros((SHIFTS, DIAG_LANES), jnp.float32)
        for b in range(NUM_BUCKETS):
            acc = jnp.where(bucket == b, rel_bias_ref[b, h], acc)
        out_ref[h] = acc


BAND = 128
C_LO_OFF = 1024
C_HI_OFF = 3008
COL_TILES = KLEN // 128


def _fill_const(stage, ct0, ct1, cvec):
    def tile_body(ct, carry):
        col = ct * 128
        for sl in range(8):
            for k in range(8):
                stage[sl, pl.ds(col + k * 16, 16)] = cvec
        return carry

    lax.fori_loop(ct0, ct1, tile_body, 0)


def _assemble_band(diag_v, stage, g, prev_bt0, clo):
    gh = g & (GROUPS_PER_HEAD - 1)
    i0 = gh * 8
    st0 = (QLEN - 1) - i0
    bt0 = jnp.maximum((i0 - (BAND - 1)) >> 7, 0)
    bt1 = jnp.minimum((i0 + 7 + (BAND - 1)) >> 7, COL_TILES - 1)

    _fill_const(stage, prev_bt0, bt0, clo)

    def band_body(ct, carry):
        col = ct * 128
        for sl in range(8):
            start = st0 - sl
            sft = start & (SHIFTS - 1)
            base = start - sft
            src = pl.multiple_of(base + col, SHIFTS)
            vals = [
                diag_v[sft, pl.ds(pl.multiple_of(src + u * 16, SHIFTS), 16)]
                for u in range(8)
            ]
            for u in range(8):
                stage[sl, pl.ds(col + u * 16, 16)] = vals[u]
        return carry

    lax.fori_loop(bt0, bt1 + 1, band_body, 0)
    return bt0


def _fanout_body(diag_hbm, out_hbm, diag_v, stage0, stage1, sem):
    wid = lax.axis_index("s") * 2 + lax.axis_index("c")
    g_lo = wid * GROUPS_PER_WORKER
    h_lo = g_lo // GROUPS_PER_HEAD
    h_hi = (g_lo + GROUPS_PER_WORKER - 1) // GROUPS_PER_HEAD

    def head_body(h, carry):
        g0 = jnp.maximum(g_lo, h * GROUPS_PER_HEAD)
        g1 = jnp.minimum(g_lo + GROUPS_PER_WORKER, (h + 1) * GROUPS_PER_HEAD)
        pltpu.sync_copy(diag_hbm.at[h], diag_v)
        clo = diag_v[0, pl.ds(C_LO_OFF, 16)]
        chi = diag_v[0, pl.ds(C_HI_OFF, 16)]
        _fill_const(stage0, 0, COL_TILES, chi)
        _fill_const(stage1, 0, COL_TILES, chi)

        def pair_body(p, carry2):
            p0, p1 = carry2
            ga = 2 * p
            gb = 2 * p + 1
            np0 = _assemble_band(diag_v, stage0, ga, p0, clo)
            cp_a = pltpu.make_async_copy(
                stage0, out_hbm.at[pl.ds(8 * ga, 8), :], sem
            )
            cp_a.start()
            np1 = _assemble_band(diag_v, stage1, gb, p1, clo)
            cp_b = pltpu.make_async_copy(
                stage1, out_hbm.at[pl.ds(8 * gb, 8), :], sem
            )
            cp_b.start()
            cp_a.wait()
            cp_b.wait()
            return (np0, np1)

        lax.fori_loop(
            g0 >> 1, g1 >> 1, pair_body, (jnp.int32(0), jnp.int32(0))
        )
        return carry

    lax.fori_loop(h_lo, h_hi + 1, head_body, 0)


def kernel(query, key, rel_bias):
    batch_size = query.shape[0]

    diag16 = pl.pallas_call(
        _diag_table_kernel,
        out_shape=jax.ShapeDtypeStruct((NUM_HEADS, SHIFTS, DIAG_LANES), jnp.float32),
    )(rel_bias)

    fanout = pl.kernel(
        _fanout_body,
        out_type=jax.ShapeDtypeStruct((NUM_HEADS * QLEN, KLEN), jnp.float32),
        mesh=plsc.VectorSubcoreMesh(core_axis_name="c", subcore_axis_name="s"),
        scratch_types=[
            pltpu.VMEM((SHIFTS, DIAG_LANES), jnp.float32),
            pltpu.VMEM((8, KLEN), jnp.float32),
            pltpu.VMEM((8, KLEN), jnp.float32),
            pltpu.SemaphoreType.DMA,
        ],
    )
    out_flat = fanout(diag16)
    out = out_flat.reshape(1, NUM_HEADS, QLEN, KLEN)
    return jnp.broadcast_to(out, (batch_size, NUM_HEADS, QLEN, KLEN))


# speedup vs baseline: 4.9871x; 1.1695x over previous
import functools
import math

import jax
import jax.numpy as jnp
from jax import lax
from jax.experimental import pallas as pl
from jax.experimental.pallas import tpu as pltpu
from jax.experimental.pallas import tpu_sc as plsc

NUM_HEADS = 12
NUM_BUCKETS = 32
MAX_DISTANCE = 128
QLEN = 2048
KLEN = 2048
SHIFTS = 16
NUM_WORKERS = 32
GROUPS = (NUM_HEADS * QLEN) // 8
GROUPS_PER_WORKER = GROUPS // NUM_WORKERS
GROUPS_PER_HEAD = QLEN // 8

BAND = 91
T0 = 1808
DIAG_LANES = 512
C_LO_OFF = 32
C_HI_OFF = 384
COL_TILES = KLEN // 128


def _diag_table_kernel(rel_bias_ref, out_ref):
    s = lax.broadcasted_iota(jnp.int32, (SHIFTS, DIAG_LANES), 0)
    u = lax.broadcasted_iota(jnp.int32, (SHIFTS, DIAG_LANES), 1)
    t = u + s + T0
    n = (QLEN - 1) - t
    half = NUM_BUCKETS // 2
    max_exact = half // 2
    ret = jnp.where(n < 0, half, 0)
    na = jnp.abs(n)
    is_small = na < max_exact
    nf = jnp.maximum(na.astype(jnp.float32), 1.0) / max_exact
    val_if_large = max_exact + (
        jnp.log(nf) / math.log(MAX_DISTANCE / max_exact) * (half - max_exact)
    ).astype(jnp.int32)
    val_if_large = jnp.minimum(val_if_large, half - 1)
    bucket = ret + jnp.where(is_small, na, val_if_large)
    for h in range(NUM_HEADS):
        acc = jnp.zeros((SHIFTS, DIAG_LANES), jnp.float32)
        for b in range(NUM_BUCKETS):
            acc = jnp.where(bucket == b, rel_bias_ref[b, h], acc)
        out_ref[h] = acc


def _fill_const(stage, ct0, ct1, cvec):
    def tile_body(ct, carry):
        col = ct * 128
        for sl in range(8):
            for k in range(8):
                stage[sl, pl.ds(col + k * 16, 16)] = cvec
        return carry

    lax.fori_loop(ct0, ct1, tile_body, 0)


def _assemble_band(diag_v, stage, g, prev_bt0, clo):
    gh = g & (GROUPS_PER_HEAD - 1)
    i0 = gh * 8
    st0 = (QLEN - 1) - i0
    bt0 = jnp.maximum((i0 - (BAND - 1)) >> 7, 0)
    bt1 = jnp.minimum((i0 + 7 + (BAND - 1)) >> 7, COL_TILES - 1)

    _fill_const(stage, prev_bt0, bt0, clo)

    def band_body(ct, carry):
        col = ct * 128
        for sl in range(8):
            start = st0 - sl
            sft = start & (SHIFTS - 1)
            base = start - sft
            src = pl.multiple_of(base + col - T0, SHIFTS)
            vals = [
                diag_v[sft, pl.ds(pl.multiple_of(src + u * 16, SHIFTS), 16)]
                for u in range(8)
            ]
            for u in range(8):
                stage[sl, pl.ds(col + u * 16, 16)] = vals[u]
        return carry

    lax.fori_loop(bt0, bt1 + 1, band_body, 0)
    return bt0


def _fanout_body(diag_hbm, out_hbm, diag_v, stage0, stage1, stage2, stage3, sem):
    wid = lax.axis_index("s") * 2 + lax.axis_index("c")
    g_lo = wid * GROUPS_PER_WORKER
    h_lo = g_lo // GROUPS_PER_HEAD
    h_hi = (g_lo + GROUPS_PER_WORKER - 1) // GROUPS_PER_HEAD
    stages = (stage0, stage1, stage2, stage3)

    def head_body(h, carry):
        g0 = jnp.maximum(g_lo, h * GROUPS_PER_HEAD)
        g1 = jnp.minimum(g_lo + GROUPS_PER_WORKER, (h + 1) * GROUPS_PER_HEAD)
        pltpu.sync_copy(diag_hbm.at[h], diag_v)
        clo = diag_v[0, pl.ds(C_LO_OFF, 16)]
        chi = diag_v[0, pl.ds(C_HI_OFF, 16)]
        for st in stages:
            _fill_const(st, 0, COL_TILES, chi)

        def quad_body(q, carry2):
            new_prev = []
            copies = []
            for k in range(4):
                g = 4 * q + k
                new_prev.append(
                    _assemble_band(diag_v, stages[k], g, carry2[k], clo)
                )
                cp = pltpu.make_async_copy(
                    stages[k], out_hbm.at[pl.ds(8 * g, 8), :], sem
                )
                cp.start()
                copies.append(cp)
            for cp in copies:
                cp.wait()
            return tuple(new_prev)

        lax.fori_loop(
            g0 >> 2, g1 >> 2, quad_body, tuple(jnp.int32(0) for _ in range(4))
        )
        return carry

    lax.fori_loop(h_lo, h_hi + 1, head_body, 0)


def kernel(query, key, rel_bias):
    batch_size = query.shape[0]

    diag16 = pl.pallas_call(
        _diag_table_kernel,
        out_shape=jax.ShapeDtypeStruct((NUM_HEADS, SHIFTS, DIAG_LANES), jnp.float32),
    )(rel_bias)

    fanout = pl.kernel(
        _fanout_body,
        out_type=jax.ShapeDtypeStruct((NUM_HEADS * QLEN, KLEN), jnp.float32),
        mesh=plsc.VectorSubcoreMesh(core_axis_name="c", subcore_axis_name="s"),
        scratch_types=[
            pltpu.VMEM((SHIFTS, DIAG_LANES), jnp.float32),
            pltpu.VMEM((8, KLEN), jnp.float32),
            pltpu.VMEM((8, KLEN), jnp.float32),
            pltpu.VMEM((8, KLEN), jnp.float32),
            pltpu.VMEM((8, KLEN), jnp.float32),
            pltpu.SemaphoreType.DMA,
        ],
    )
    out_flat = fanout(diag16)
    out = out_flat.reshape(1, NUM_HEADS, QLEN, KLEN)
    return jnp.broadcast_to(out, (batch_size, NUM_HEADS, QLEN, KLEN))
